# Initial kernel scaffold; baseline (speedup 1.0000x reference)
import jax
import jax.numpy as jnp
from jax.experimental import pallas as pl


def kernel(x, edge_index, batch, W1, b1, W2, b2, Wc, bc):
    raise NotImplementedError("write your pallas kernel here")


# speedup vs baseline: 4.7679x; 4.7679x over previous
import functools

import jax
import jax.numpy as jnp
from jax import lax
from jax.experimental import pallas as pl
from jax.experimental.pallas import tpu as pltpu
from jax.experimental.pallas import tpu_sc as plsc

N = 100000
NUM_GRAPHS = 64
IN_C = 8
HID = 32
HHALF = 16
REP = 128

NPAD = 100352
BN = 2048
NBLK = NPAD // BN
SLICE = NPAD // 16

E = 1600000
RW = 128
R = 12544
EPAD = R * RW
NPADROWS = 352
ROWS_PER_SUB = R // 16
CE = 64
CROWS2 = R * 2
ROWS_PER_SUB2 = CROWS2 // 16
NCHUNK = ROWS_PER_SUB2
ROWS_PER_W = R // 32
DCHUNK = 8
NDCHUNK = ROWS_PER_W // DCHUNK

_mesh = plsc.VectorSubcoreMesh(core_axis_name="c", subcore_axis_name="s")


@functools.partial(
    pl.kernel,
    out_type=jax.ShapeDtypeStruct((2, NPAD), jnp.float32),
    mesh=_mesh,
    scratch_types=[
        pltpu.VMEM((DCHUNK, RW), jnp.int32),
        pltpu.VMEM((RW,), jnp.float32),
        pltpu.VMEM((SLICE,), jnp.float32),
        pltpu.VMEM_SHARED((NPAD,), jnp.float32),
        pltpu.SemaphoreType.DMA,
    ],
)
def _deg_kernel(dst_hbm, out_hbm, didx, ones_v, zbuf, deg_sh, sem):
  c = lax.axis_index("c")
  s = lax.axis_index("s")
  w = s * 2 + c

  def _fill(i, _):
    zbuf[pl.ds(i * 16, 16)] = jnp.zeros((16,), jnp.float32)
    return 0

  lax.fori_loop(0, SLICE // 16, _fill, 0)
  for i in range(RW // 16):
    ones_v[pl.ds(i * 16, 16)] = jnp.ones((16,), jnp.float32)
  pltpu.sync_copy(zbuf, deg_sh.at[pl.ds(s * SLICE, SLICE)])
  plsc.subcore_barrier()

  row0 = w * ROWS_PER_W

  def _dchunk(k, _):
    pltpu.sync_copy(dst_hbm.at[pl.ds(row0 + k * DCHUNK, DCHUNK)], didx)
    descs = []
    for j in range(DCHUNK):
      descs.append(
          pltpu.async_copy(ones_v, deg_sh.at[didx.at[j]], sem, add=True))
    for d in descs:
      d.wait()
    return 0

  lax.fori_loop(0, NDCHUNK, _dchunk, 0)
  plsc.subcore_barrier()
  pltpu.sync_copy(deg_sh.at[pl.ds(s * SLICE, SLICE)],
                  out_hbm.at[c, pl.ds(s * SLICE, SLICE)])


@functools.partial(
    pl.kernel,
    out_type=jax.ShapeDtypeStruct((2 * NPAD * HHALF,), jnp.float32),
    mesh=_mesh,
    scratch_types=[
        pltpu.VMEM((1, CE), jnp.int32),
        pltpu.VMEM((1, CE), jnp.int32),
        pltpu.VMEM((CE, REP), jnp.float32),
        pltpu.VMEM((CE * HHALF,), jnp.float32),
        pltpu.VMEM((CE * HHALF // 128, 128), jnp.int32),
        pltpu.VMEM((CE * HHALF,), jnp.float32),
        pltpu.VMEM_SHARED((NPAD * HHALF,), jnp.float32),
        pltpu.SemaphoreType.DMA,
        pltpu.SemaphoreType.DMA,
    ],
)
def _prop_kernel(src_hbm, exp_hbm, tbl_hbm, out_hbm,
                 sidx, didx, rows, comp1, sxi, zb1, acc1, gsem, ssem):
  c = lax.axis_index("c")
  s = lax.axis_index("s")
  nsl = SLICE * HHALF
  ncp = CE * HHALF
  def _zfill(i, _):
    zb1[pl.ds(i * 16, 16)] = jnp.zeros((16,), jnp.float32)
    return 0

  lax.fori_loop(0, ncp // 16, _zfill, 0)

  def _zero(k, _):
    pltpu.sync_copy(zb1, acc1.at[pl.ds(s * nsl + k * ncp, ncp)])
    return 0

  lax.fori_loop(0, nsl // ncp, _zero, 0)
  plsc.subcore_barrier()

  row0 = s * ROWS_PER_SUB2

  def _chunk(k, _):
    base = row0 + k
    pltpu.sync_copy(src_hbm.at[pl.ds(base, 1)], sidx)
    pltpu.sync_copy(exp_hbm.at[pl.ds(base * 8, 8)], sxi)
    pltpu.async_copy(tbl_hbm.at[sidx.at[0]], rows, gsem).wait()

    @pl.when(c == 0)
    def _():
      for e in range(CE):
        comp1[pl.ds(e * HHALF, HHALF)] = rows[e, pl.ds(0, HHALF)]

    @pl.when(c == 1)
    def _():
      for e in range(CE):
        comp1[pl.ds(e * HHALF, HHALF)] = rows[e, pl.ds(HHALF, HHALF)]

    sds = []
    for j in range(ncp // 128):
      sds.append(
          pltpu.async_copy(comp1.at[pl.ds(j * 128, 128)],
                           acc1.at[sxi.at[j]], ssem, add=True))
    for d in sds:
      d.wait()
    return 0

  lax.fori_loop(0, NCHUNK, _chunk, 0)
  plsc.subcore_barrier()
  pltpu.sync_copy(acc1.at[pl.ds(s * nsl, nsl)],
                  out_hbm.at[pl.ds(c * NPAD * HHALF + s * nsl, nsl)])


def _tc1_body(xref, d0ref, d1ref, wref, oref):
  deg = d0ref[0, 0, :] + d1ref[0, 0, :] + 1.0
  dinv = lax.rsqrt(deg)
  h = jnp.dot(xref[...], wref[...], preferred_element_type=jnp.float32)
  hp = h * dinv[:, None]
  oref[...] = jnp.concatenate([hp, hp, hp, hp], axis=1)


def _tc2_body(aref, href, d0ref, d1ref, wref, bref, oref):
  deg = d0ref[0, 0, :] + d1ref[0, 0, :] + 1.0
  dinv = lax.rsqrt(deg)
  acc = jnp.concatenate([aref[0], aref[1]], axis=1)
  hp = href[:, :HID]
  h1 = jnp.maximum(dinv[:, None] * (acc + hp) + bref[...], 0.0)
  t = jnp.dot(h1, wref[...], preferred_element_type=jnp.float32)
  h2p = t * dinv[:, None]
  oref[...] = jnp.concatenate([h2p, h2p, h2p, h2p], axis=1)


def _tc3_body(aref, href, d0ref, d1ref, bref, segref, wcref, bcref, oref,
              accum):
  i = pl.program_id(0)

  @pl.when(i == 0)
  def _():
    accum[...] = jnp.zeros_like(accum)

  deg = d0ref[0, 0, :] + d1ref[0, 0, :] + 1.0
  dinv = lax.rsqrt(deg)
  acc = jnp.concatenate([aref[0], aref[1]], axis=1)
  hp = href[:, :HID]
  h2 = jnp.maximum(dinv[:, None] * (acc + hp) + bref[...], 0.0)
  hcat = jnp.concatenate([h2, jnp.ones((BN, 1), jnp.float32)], axis=1)
  seg = segref[0, 0, :]
  onehot = (seg[:, None] == lax.broadcasted_iota(jnp.int32, (BN, NUM_GRAPHS),
                                                 1)).astype(jnp.float32)
  accum[...] += lax.dot_general(onehot, hcat, (((0,), (0,)), ((), ())),
                                preferred_element_type=jnp.float32)

  @pl.when(i == NBLK - 1)
  def _():
    sums = accum[:, :HID]
    cnt = jnp.maximum(accum[:, HID:HID + 1], 1.0)
    pooled = sums / cnt
    oref[...] = jnp.dot(pooled, wcref[...],
                        preferred_element_type=jnp.float32) + bcref[...]


def _deg_specs():
  blk = pl.BlockSpec((1, 1, BN), lambda i: (i, 0, 0))
  blk2 = pl.BlockSpec((1, 1, BN), lambda i: (i + NBLK, 0, 0))
  return blk, blk2


def _tc1(x_pad, deg3, w1):
  d0, d1 = _deg_specs()
  return pl.pallas_call(
      _tc1_body,
      grid=(NBLK,),
      in_specs=[
          pl.BlockSpec((BN, IN_C), lambda i: (i, 0)),
          d0, d1,
          pl.BlockSpec((IN_C, HID), lambda i: (0, 0)),
      ],
      out_specs=pl.BlockSpec((BN, REP), lambda i: (i, 0)),
      out_shape=jax.ShapeDtypeStruct((NPAD, REP), jnp.float32),
  )(x_pad, deg3, deg3, w1)


def _tc2(acc, hst, deg3, w2, b1):
  d0, d1 = _deg_specs()
  stk = pl.BlockSpec((2, BN, HHALF), lambda i: (0, i, 0))
  return pl.pallas_call(
      _tc2_body,
      grid=(NBLK,),
      in_specs=[
          stk,
          pl.BlockSpec((BN, REP), lambda i: (i, 0)),
          d0, d1,
          pl.BlockSpec((HID, HID), lambda i: (0, 0)),
          pl.BlockSpec((1, HID), lambda i: (0, 0)),
      ],
      out_specs=pl.BlockSpec((BN, REP), lambda i: (i, 0)),
      out_shape=jax.ShapeDtypeStruct((NPAD, REP), jnp.float32),
  )(acc, hst, deg3, deg3, w2, b1)


def _tc3(acc, hst, deg3, b2, seg3, wc, bc):
  d0, d1 = _deg_specs()
  stk = pl.BlockSpec((2, BN, HHALF), lambda i: (0, i, 0))
  return pl.pallas_call(
      _tc3_body,
      grid=(NBLK,),
      in_specs=[
          stk,
          pl.BlockSpec((BN, REP), lambda i: (i, 0)),
          d0, d1,
          pl.BlockSpec((1, HID), lambda i: (0, 0)),
          pl.BlockSpec((1, 1, BN), lambda i: (i, 0, 0)),
          pl.BlockSpec((HID, 2), lambda i: (0, 0)),
          pl.BlockSpec((1, 2), lambda i: (0, 0)),
      ],
      out_specs=pl.BlockSpec((NUM_GRAPHS, 2), lambda i: (0, 0)),
      out_shape=jax.ShapeDtypeStruct((NUM_GRAPHS, 2), jnp.float32),
      scratch_shapes=[pltpu.VMEM((NUM_GRAPHS, HID + 1), jnp.float32)],
  )(acc, hst, deg3, deg3, b2, seg3, wc, bc)


def kernel(x, edge_index, batch, W1, b1, W2, b2, Wc, bc):
  src = edge_index[0]
  dst = edge_index[1]
  pad_idx = N + (jnp.arange(EPAD - E, dtype=jnp.int32) % NPADROWS)
  srcp = jnp.concatenate([src, pad_idx])
  dstp = jnp.concatenate([dst, pad_idx])
  dst2d = dstp.reshape(R, RW)
  src64 = srcp.reshape(CROWS2, CE)
  exp2d = ((dstp[:, None] * HHALF) + jnp.arange(HHALF, dtype=jnp.int32)
           ).reshape(EPAD * HHALF // 128, 128)

  x_pad = jnp.pad(x, ((0, NPAD - N), (0, 0)))
  seg3 = jnp.pad(batch, (0, NPAD - N),
                 constant_values=NUM_GRAPHS).reshape(NBLK, 1, BN)

  deg2 = _deg_kernel(dst2d)
  deg3 = deg2.reshape(2 * NBLK, 1, BN)
  hst1 = _tc1(x_pad, deg3, W1)
  acc1 = _prop_kernel(src64, exp2d, hst1).reshape(2, NPAD, HHALF)
  hst2 = _tc2(acc1, hst1, deg3, W2, b1.reshape(1, HID))
  acc2 = _prop_kernel(src64, exp2d, hst2).reshape(2, NPAD, HHALF)
  out = _tc3(acc2, hst2, deg3, b2.reshape(1, HID), seg3, Wc,
             bc.reshape(1, 2))
  return out
